# baseline (speedup 1.0000x reference)
import functools
import jax
import jax.numpy as jnp
from jax import lax
from jax.experimental import pallas as pl
from jax.experimental.pallas import tpu as pltpu
from jax.experimental.pallas import tpu_sc as plsc

B = 4096
L = 200
D = 64
TOTAL = B * L

_info = plsc.get_sparse_core_info()
NC = _info.num_cores
NS = _info.num_subcores
NW = NC * NS
PER_W = TOTAL // NW
CHUNK = 512
N_CHUNKS = PER_W // CHUNK

_mesh = plsc.VectorSubcoreMesh(core_axis_name="c", subcore_axis_name="s")


@functools.partial(
    pl.kernel,
    mesh=_mesh,
    out_type=jax.ShapeDtypeStruct((TOTAL, D), jnp.float32),
    scratch_types=[
        pltpu.VMEM((CHUNK,), jnp.int32),
        pltpu.VMEM((CHUNK, D), jnp.float32),
        pltpu.SemaphoreType.DMA,
    ],
    compiler_params=pltpu.CompilerParams(use_tc_tiling_on_sc=False),
)
def _gather_kernel(idx_hbm, table_hbm, out_hbm, idx_v, rows_v, sem):
    wid = lax.axis_index("s") * NC + lax.axis_index("c")
    base = wid * PER_W

    def body(g, carry):
        off = base + g * CHUNK
        pltpu.sync_copy(idx_hbm.at[pl.ds(off, CHUNK)], idx_v)
        pltpu.async_copy(table_hbm.at[idx_v], rows_v, sem).wait()
        pltpu.sync_copy(rows_v, out_hbm.at[pl.ds(off, CHUNK)])
        return carry

    lax.fori_loop(0, N_CHUNKS, body, 0)


def _mask_body(idx_ref, out_ref):
    out_ref[...] = (idx_ref[...] > 0).astype(jnp.int32)


_mask = pl.pallas_call(
    _mask_body,
    out_shape=jax.ShapeDtypeStruct((6400, 128), jnp.int32),
    grid=(8,),
    in_specs=[pl.BlockSpec((800, 128), lambda i: (i, 0))],
    out_specs=pl.BlockSpec((800, 128), lambda i: (i, 0)),
)


@jax.jit
def kernel(word_idx, tune_pre_trained, table):
    idx_flat = word_idx.reshape(TOTAL).astype(jnp.int32)
    rows = _gather_kernel(idx_flat, table)
    static_emb = rows.reshape(B, L, D)
    mask = _mask(idx_flat.reshape(6400, 128))
    bottom_existence = mask.reshape(B, L, 1).astype(jnp.bool_)
    return (static_emb, bottom_existence)


# speedup vs baseline: 1.0378x; 1.0378x over previous
import functools
import jax
import jax.numpy as jnp
from jax import lax
from jax.experimental import pallas as pl
from jax.experimental.pallas import tpu as pltpu
from jax.experimental.pallas import tpu_sc as plsc

B = 4096
L = 200
D = 64
TOTAL = B * L

_info = plsc.get_sparse_core_info()
NC = _info.num_cores
NS = _info.num_subcores
NW = NC * NS
PER_W = TOTAL // NW
CHUNK = 800
N_CHUNKS = PER_W // CHUNK
T_PAIRS = N_CHUNKS // 2

_mesh = plsc.VectorSubcoreMesh(core_axis_name="c", subcore_axis_name="s")


@functools.partial(
    pl.kernel,
    mesh=_mesh,
    out_type=jax.ShapeDtypeStruct((TOTAL, D), jnp.float32),
    scratch_types=[
        pltpu.VMEM((CHUNK,), jnp.int32),
        pltpu.VMEM((CHUNK,), jnp.int32),
        pltpu.VMEM((CHUNK, D), jnp.float32),
        pltpu.VMEM((CHUNK, D), jnp.float32),
        pltpu.SemaphoreType.DMA,
        pltpu.SemaphoreType.DMA,
        pltpu.SemaphoreType.DMA,
        pltpu.SemaphoreType.DMA,
    ],
    compiler_params=pltpu.CompilerParams(use_tc_tiling_on_sc=False),
)
def _gather_kernel(idx_hbm, table_hbm, out_hbm,
                   idx_a, idx_b, rows_a, rows_b,
                   gsem_a, gsem_b, wsem_a, wsem_b):
    wid = lax.axis_index("s") * NC + lax.axis_index("c")
    base = wid * PER_W

    def out_at(c):
        return out_hbm.at[pl.ds(base + c * CHUNK, CHUNK)]

    def idx_at(c):
        return idx_hbm.at[pl.ds(base + c * CHUNK, CHUNK)]

    pltpu.sync_copy(idx_at(0), idx_a)
    pltpu.async_copy(table_hbm.at[idx_a], rows_a, gsem_a)

    def body(t, carry):
        c0 = 2 * t
        c1 = c0 + 1

        @pl.when(t > 0)
        def _():
            pltpu.make_async_copy(rows_b, out_at(c1), wsem_b).wait()

        pltpu.sync_copy(idx_at(c1), idx_b)
        pltpu.async_copy(table_hbm.at[idx_b], rows_b, gsem_b)

        pltpu.make_async_copy(table_hbm.at[idx_a], rows_a, gsem_a).wait()
        pltpu.async_copy(rows_a, out_at(c0), wsem_a)

        @pl.when(t < T_PAIRS - 1)
        def _():
            pltpu.make_async_copy(rows_a, out_at(c0), wsem_a).wait()
            pltpu.sync_copy(idx_at(c0 + 2), idx_a)
            pltpu.async_copy(table_hbm.at[idx_a], rows_a, gsem_a)

        pltpu.make_async_copy(table_hbm.at[idx_b], rows_b, gsem_b).wait()
        pltpu.async_copy(rows_b, out_at(c1), wsem_b)
        return carry

    lax.fori_loop(0, T_PAIRS, body, 0)
    pltpu.make_async_copy(rows_a, out_at(N_CHUNKS - 2), wsem_a).wait()
    pltpu.make_async_copy(rows_b, out_at(N_CHUNKS - 1), wsem_b).wait()


def _mask_body(idx_ref, out_ref):
    out_ref[...] = (idx_ref[...] > 0).astype(jnp.int32)


_mask = pl.pallas_call(
    _mask_body,
    out_shape=jax.ShapeDtypeStruct((6400, 128), jnp.int32),
    grid=(8,),
    in_specs=[pl.BlockSpec((800, 128), lambda i: (i, 0))],
    out_specs=pl.BlockSpec((800, 128), lambda i: (i, 0)),
)


@jax.jit
def kernel(word_idx, tune_pre_trained, table):
    idx_flat = word_idx.reshape(TOTAL).astype(jnp.int32)
    rows = _gather_kernel(idx_flat, table)
    static_emb = rows.reshape(B, L, D)
    mask = _mask(idx_flat.reshape(6400, 128))
    bottom_existence = mask.reshape(B, L, 1).astype(jnp.bool_)
    return (static_emb, bottom_existence)
